# baseline (speedup 1.0000x reference)
import functools

import jax
import jax.numpy as jnp
import numpy as np
from jax.experimental import pallas as pl
from jax.experimental.pallas import tpu as pltpu

_EPS = 1e-5
_GROUPS = 32


def _hw_shifts(H, W):
    return tuple((kh - 1) * W + (kw - 1)
                 for kh in range(3) for kw in range(3))


def _hw_masks(D, H, W):
    h = np.arange(H)[:, None]
    w = np.arange(W)[None, :]
    ms = []
    for kh in range(3):
        for kw in range(3):
            dh, dw = kh - 1, kw - 1
            m = ((h + dh >= 0) & (h + dh < H) &
                 (w + dw >= 0) & (w + dw < W))
            m = np.broadcast_to(m[None, :, :], (D, H, W))
            ms.append(m.reshape(1, D * H * W))
    return jnp.asarray(np.stack(ms).astype(np.float32)).astype(jnp.bfloat16)


def _group_proj(C, groups, count):
    cg = C // groups
    m = np.zeros((C, C), np.float32)
    for g in range(groups):
        m[g * cg:(g + 1) * cg, g * cg:(g + 1) * cg] = 1.0 / count
    return jnp.asarray(m)


def _fold_w(w):
    Cout, C = w.shape[:2]
    wt = jnp.transpose(w.astype(jnp.bfloat16), (2, 0, 3, 4, 1))
    return wt.reshape(3, Cout, 9 * C)


def _block_kernel(x_ref, p1_ref, g1_ref, b1_ref, w1_ref, bias1_ref,
                  p2_ref, g2_ref, b2_ref, w2_ref, bias2_ref,
                  mhw_ref, wr_ref, br_ref, e1_ref, e2_ref, o_ref, *a_refs,
                  shifts, S, HW, Cin, Cout, NB):
    PAD = HW

    def gn_silu(v, p_ref, g_col, b_col):
        s1 = jnp.sum(v, axis=1, keepdims=True)
        s2 = jnp.sum(v * v, axis=1, keepdims=True)
        mean = jnp.dot(p_ref[...], s1, preferred_element_type=jnp.float32,
                       precision=jax.lax.Precision.HIGHEST)
        msq = jnp.dot(p_ref[...], s2, preferred_element_type=jnp.float32,
                      precision=jax.lax.Precision.HIGHEST)
        inv = jax.lax.rsqrt(msq - mean * mean + _EPS)
        a = (v - mean) * (inv * g_col) + b_col
        return a * (1.0 / (1.0 + jnp.exp(-a)))

    def to_cols(eye_ref, *rows):
        m = jnp.concatenate(rows, axis=0)
        cols = jax.lax.dot_general(eye_ref[...], m, (((1,), (1,)), ((), ())),
                                   preferred_element_type=jnp.float32,
                                   precision=jax.lax.Precision.HIGHEST)
        return [cols[:, i:i + 1] for i in range(len(rows))]

    def zero_pads(aj_ref):
        zpad = jnp.zeros((9 * Cout, PAD), jnp.bfloat16)
        aj_ref[:, 0:PAD] = zpad
        aj_ref[:, PAD + S:2 * PAD + S] = zpad

    def stack_taps(aj_ref, act, C):
        act_bf = act.astype(jnp.bfloat16)
        for i, off in enumerate(shifts):
            k = off % S
            if k == 0:
                r = act_bf
            else:
                r = jnp.concatenate([act_bf[:, k:], act_bf[:, :k]], axis=1)
            if off != 0:
                r = r * mhw_ref[i]
            aj_ref[i * C:(i + 1) * C, PAD:PAD + S] = r

    def conv(aj_ref, w_ref, bias_col, C):
        acc = None
        for kd in range(3):
            blk = aj_ref[0:9 * C, kd * HW:kd * HW + S]
            c = jnp.dot(w_ref[kd], blk, preferred_element_type=jnp.float32)
            acc = c if acc is None else acc + c
        return acc + bias_col

    @pl.when(pl.program_id(0) == 0)
    def _init_pads():
        for aj_ref in a_refs:
            zero_pads(aj_ref)

    g1c, b1c = to_cols(e1_ref, g1_ref[...], b1_ref[...])
    g2c, b2c, bias1c, bias2c, brc = to_cols(
        e2_ref, g2_ref[...], b2_ref[...], bias1_ref[...], bias2_ref[...],
        br_ref[...])

    for j in range(NB):
        aj_ref = a_refs[j]
        x = x_ref[j]
        act1 = gn_silu(x, p1_ref, g1c, b1c)
        stack_taps(aj_ref, act1, Cin)
        h = conv(aj_ref, w1_ref, bias1c, Cin)
        act2 = gn_silu(h, p2_ref, g2c, b2c)
        stack_taps(aj_ref, act2, Cout)
        y = conv(aj_ref, w2_ref, bias2c, Cout)
        res = jnp.dot(wr_ref[...].astype(jnp.bfloat16),
                      x.astype(jnp.bfloat16),
                      preferred_element_type=jnp.float32)
        o_ref[j] = (y + res + brc).astype(o_ref.dtype)


def kernel(x, g1, be1, w1, bias1, g2, be2, w2, bias2, wr, br):
    N, Cin, D, H, W = x.shape
    Cout = w1.shape[0]
    S = D * H * W
    HW = H * W

    xf = x.reshape(N, Cin, S)
    mhw = _hw_masks(D, H, W)
    p1 = _group_proj(Cin, _GROUPS, (Cin // _GROUPS) * S)
    p2 = _group_proj(Cout, _GROUPS, (Cout // _GROUPS) * S)
    w1f = _fold_w(w1)
    w2f = _fold_w(w2)
    g1v = g1.reshape(1, Cin)
    be1v = be1.reshape(1, Cin)
    g2v = g2.reshape(1, Cout)
    be2v = be2.reshape(1, Cout)
    bias1v = bias1.reshape(1, Cout)
    bias2v = bias2.reshape(1, Cout)
    wrf = wr.reshape(Cout, Cin)
    brv = br.reshape(1, Cout)
    e1 = jnp.asarray(np.eye(Cin, dtype=np.float32))
    e2 = jnp.asarray(np.eye(Cout, dtype=np.float32))
    NB = 2 if N % 2 == 0 else 1
    cnst = lambda *shape: pl.BlockSpec(shape, lambda n: (0,) * len(shape))
    out = pl.pallas_call(
        functools.partial(_block_kernel, shifts=_hw_shifts(H, W),
                          S=S, HW=HW, Cin=Cin, Cout=Cout, NB=NB),
        out_shape=jax.ShapeDtypeStruct((N, Cout, S), x.dtype),
        grid=(N // NB,),
        in_specs=[
            pl.BlockSpec((NB, Cin, S), lambda n: (n, 0, 0)),
            cnst(Cin, Cin),
            cnst(1, Cin),
            cnst(1, Cin),
            cnst(3, Cout, 9 * Cin),
            cnst(1, Cout),
            cnst(Cout, Cout),
            cnst(1, Cout),
            cnst(1, Cout),
            cnst(3, Cout, 9 * Cout),
            cnst(1, Cout),
            cnst(9, 1, S),
            cnst(Cout, Cin),
            cnst(1, Cout),
            cnst(Cin, Cin),
            cnst(Cout, Cout),
        ],
        out_specs=pl.BlockSpec((NB, Cout, S), lambda n: (n, 0, 0)),
        scratch_shapes=[pltpu.VMEM((9 * Cout, S + 2 * HW), jnp.bfloat16)
                        for _ in range(NB)],
        compiler_params=pltpu.CompilerParams(
            dimension_semantics=("parallel",)),
    )(xf, p1, g1v, be1v, w1f, bias1v, p2, g2v, be2v, w2f, bias2v,
      mhw, wrf, brv, e1, e2)

    return out.reshape(N, Cout, D, H, W)


# speedup vs baseline: 1.0313x; 1.0313x over previous
import functools

import jax
import jax.numpy as jnp
import numpy as np
from jax.experimental import pallas as pl
from jax.experimental.pallas import tpu as pltpu

_EPS = 1e-5
_GROUPS = 32


def _hw_shifts(H, W):
    return tuple((kh - 1) * W + (kw - 1)
                 for kh in range(3) for kw in range(3))


def _hw_masks(D, H, W):
    h = np.arange(H)[:, None]
    w = np.arange(W)[None, :]
    ms = []
    for kh in range(3):
        for kw in range(3):
            dh, dw = kh - 1, kw - 1
            m = ((h + dh >= 0) & (h + dh < H) &
                 (w + dw >= 0) & (w + dw < W))
            m = np.broadcast_to(m[None, :, :], (D, H, W))
            ms.append(m.reshape(1, D * H * W))
    return jnp.asarray(np.stack(ms).astype(np.float32)).astype(jnp.bfloat16)


def _group_proj(C, groups, count):
    cg = C // groups
    m = np.zeros((C, C), np.float32)
    for g in range(groups):
        m[g * cg:(g + 1) * cg, g * cg:(g + 1) * cg] = 1.0 / count
    return jnp.asarray(m)


def _fold_w(w):
    Cout, C = w.shape[:2]
    wt = jnp.transpose(w.astype(jnp.bfloat16), (2, 0, 3, 4, 1))
    return wt.reshape(3, Cout, 9 * C)


def _block_kernel(x_ref, p1_ref, g1_ref, b1_ref, w1_ref, bias1_ref,
                  p2_ref, g2_ref, b2_ref, w2_ref, bias2_ref,
                  mhw_ref, wr_ref, br_ref, e1_ref, e2_ref, o_ref, *a_refs,
                  shifts, S, HW, Cin, Cout, NB):
    PAD = HW

    def gn_silu(v, p_ref, g_col, b_col):
        s1 = jnp.sum(v, axis=1, keepdims=True)
        s2 = jnp.sum(v * v, axis=1, keepdims=True)
        mean = jnp.dot(p_ref[...], s1, preferred_element_type=jnp.float32)
        msq = jnp.dot(p_ref[...], s2, preferred_element_type=jnp.float32)
        inv = jax.lax.rsqrt(msq - mean * mean + _EPS)
        a = (v - mean) * (inv * g_col) + b_col
        return a * (1.0 / (1.0 + jnp.exp(-a)))

    def to_cols(eye_ref, *rows):
        m = jnp.concatenate(rows, axis=0)
        cols = jax.lax.dot_general(eye_ref[...], m, (((1,), (1,)), ((), ())),
                                   preferred_element_type=jnp.float32,
                                   precision=jax.lax.Precision.HIGHEST)
        return [cols[:, i:i + 1] for i in range(len(rows))]

    def zero_pads(aj_ref):
        zpad = jnp.zeros((9 * Cout, PAD), jnp.bfloat16)
        aj_ref[:, 0:PAD] = zpad
        aj_ref[:, PAD + S:2 * PAD + S] = zpad

    def stack_taps(aj_ref, act, C):
        act_bf = act.astype(jnp.bfloat16)
        for i, off in enumerate(shifts):
            k = off % S
            if k == 0:
                r = act_bf
            else:
                r = jnp.concatenate([act_bf[:, k:], act_bf[:, :k]], axis=1)
            if off != 0:
                r = r * mhw_ref[i]
            aj_ref[i * C:(i + 1) * C, PAD:PAD + S] = r

    def conv(aj_ref, w_ref, bias_col, C):
        acc = None
        for kd in range(3):
            blk = aj_ref[0:9 * C, kd * HW:kd * HW + S]
            c = jnp.dot(w_ref[kd], blk, preferred_element_type=jnp.float32)
            acc = c if acc is None else acc + c
        return acc + bias_col

    @pl.when(pl.program_id(0) == 0)
    def _init_pads():
        for aj_ref in a_refs:
            zero_pads(aj_ref)

    g1c, b1c = to_cols(e1_ref, g1_ref[...], b1_ref[...])
    g2c, b2c, bias1c, bias2c, brc = to_cols(
        e2_ref, g2_ref[...], b2_ref[...], bias1_ref[...], bias2_ref[...],
        br_ref[...])

    for j in range(NB):
        aj_ref = a_refs[j]
        x = x_ref[j]
        act1 = gn_silu(x, p1_ref, g1c, b1c)
        stack_taps(aj_ref, act1, Cin)
        h = conv(aj_ref, w1_ref, bias1c, Cin)
        act2 = gn_silu(h, p2_ref, g2c, b2c)
        stack_taps(aj_ref, act2, Cout)
        y = conv(aj_ref, w2_ref, bias2c, Cout)
        res = jnp.dot(wr_ref[...].astype(jnp.bfloat16),
                      x.astype(jnp.bfloat16),
                      preferred_element_type=jnp.float32)
        o_ref[j] = (y + res + brc).astype(o_ref.dtype)


def kernel(x, g1, be1, w1, bias1, g2, be2, w2, bias2, wr, br):
    N, Cin, D, H, W = x.shape
    Cout = w1.shape[0]
    S = D * H * W
    HW = H * W

    xf = x.reshape(N, Cin, S)
    mhw = _hw_masks(D, H, W)
    p1 = _group_proj(Cin, _GROUPS, (Cin // _GROUPS) * S)
    p2 = _group_proj(Cout, _GROUPS, (Cout // _GROUPS) * S)
    w1f = _fold_w(w1)
    w2f = _fold_w(w2)
    g1v = g1.reshape(1, Cin)
    be1v = be1.reshape(1, Cin)
    g2v = g2.reshape(1, Cout)
    be2v = be2.reshape(1, Cout)
    bias1v = bias1.reshape(1, Cout)
    bias2v = bias2.reshape(1, Cout)
    wrf = wr.reshape(Cout, Cin)
    brv = br.reshape(1, Cout)
    e1 = jnp.asarray(np.eye(Cin, dtype=np.float32))
    e2 = jnp.asarray(np.eye(Cout, dtype=np.float32))
    NB = 2 if N % 2 == 0 else 1
    cnst = lambda *shape: pl.BlockSpec(shape, lambda n: (0,) * len(shape))
    out = pl.pallas_call(
        functools.partial(_block_kernel, shifts=_hw_shifts(H, W),
                          S=S, HW=HW, Cin=Cin, Cout=Cout, NB=NB),
        out_shape=jax.ShapeDtypeStruct((N, Cout, S), x.dtype),
        grid=(N // NB,),
        in_specs=[
            pl.BlockSpec((NB, Cin, S), lambda n: (n, 0, 0)),
            cnst(Cin, Cin),
            cnst(1, Cin),
            cnst(1, Cin),
            cnst(3, Cout, 9 * Cin),
            cnst(1, Cout),
            cnst(Cout, Cout),
            cnst(1, Cout),
            cnst(1, Cout),
            cnst(3, Cout, 9 * Cout),
            cnst(1, Cout),
            cnst(9, 1, S),
            cnst(Cout, Cin),
            cnst(1, Cout),
            cnst(Cin, Cin),
            cnst(Cout, Cout),
        ],
        out_specs=pl.BlockSpec((NB, Cout, S), lambda n: (n, 0, 0)),
        scratch_shapes=[pltpu.VMEM((9 * Cout, S + 2 * HW), jnp.bfloat16)
                        for _ in range(NB)],
        compiler_params=pltpu.CompilerParams(
            dimension_semantics=("parallel",)),
    )(xf, p1, g1v, be1v, w1f, bias1v, p2, g2v, be2v, w2f, bias2v,
      mhw, wrf, brv, e1, e2)

    return out.reshape(N, Cout, D, H, W)


# speedup vs baseline: 1.1688x; 1.1333x over previous
import functools

import jax
import jax.numpy as jnp
import numpy as np
from jax.experimental import pallas as pl
from jax.experimental.pallas import tpu as pltpu

_EPS = 1e-5
_GROUPS = 32


def _hw_shifts(H, W):
    return tuple((kh - 1) * W + (kw - 1)
                 for kh in range(3) for kw in range(3))


def _hw_masks(D, H, W):
    h = np.arange(H)[:, None]
    w = np.arange(W)[None, :]
    ms = []
    for kh in range(3):
        for kw in range(3):
            dh, dw = kh - 1, kw - 1
            m = ((h + dh >= 0) & (h + dh < H) &
                 (w + dw >= 0) & (w + dw < W))
            m = np.broadcast_to(m[None, :, :], (D, H, W))
            ms.append(m.reshape(1, D * H * W))
    return jnp.asarray(np.stack(ms).astype(np.float32)).astype(jnp.bfloat16)


def _group_proj(C, groups, count):
    cg = C // groups
    m = np.zeros((C, C), np.float32)
    for g in range(groups):
        m[g * cg:(g + 1) * cg, g * cg:(g + 1) * cg] = 1.0 / count
    return jnp.asarray(m)


def _fold_w(w):
    Cout, C = w.shape[:2]
    wt = jnp.transpose(w.astype(jnp.bfloat16), (2, 0, 3, 4, 1))
    return wt.reshape(3, Cout, 9 * C)


def _block_kernel(x_ref, p1_ref, g1_ref, b1_ref, w1_ref, bias1_ref,
                  p2_ref, g2_ref, b2_ref, w2_ref, bias2_ref,
                  mhw_ref, wp_ref, br_ref, e1_ref, e2_ref, o_ref, *a_refs,
                  shifts, S, HW, Cin, Cout, NB):
    PAD = HW

    def gn_silu(v, p_ref, g_col, b_col):
        s1 = jnp.sum(v, axis=1, keepdims=True)
        s2 = jnp.sum(v * v, axis=1, keepdims=True)
        mean = jnp.dot(p_ref[...], s1, preferred_element_type=jnp.float32)
        msq = jnp.dot(p_ref[...], s2, preferred_element_type=jnp.float32)
        inv = jax.lax.rsqrt(msq - mean * mean + _EPS)
        a = (v - mean) * (inv * g_col) + b_col
        return a * (1.0 / (1.0 + jnp.exp(-a)))

    def to_cols(eye_ref, *rows):
        m = jnp.concatenate(rows, axis=0)
        cols = jax.lax.dot_general(eye_ref[...], m, (((1,), (1,)), ((), ())),
                                   preferred_element_type=jnp.float32,
                                   precision=jax.lax.Precision.HIGHEST)
        return [cols[:, i:i + 1] for i in range(len(rows))]

    def zero_pads(aj_ref):
        zpad = jnp.zeros((9 * Cout, PAD), jnp.bfloat16)
        aj_ref[:, 0:PAD] = zpad
        aj_ref[:, PAD + S:2 * PAD + S] = zpad

    def stack_taps(aj_ref, act, C):
        act_bf = act.astype(jnp.bfloat16)
        for i, off in enumerate(shifts):
            k = off % S
            if k == 0:
                r = act_bf
            else:
                r = jnp.concatenate([act_bf[:, k:], act_bf[:, :k]], axis=1)
            if off != 0:
                r = r * mhw_ref[i]
            aj_ref[i * C:(i + 1) * C, PAD:PAD + S] = r

    def conv(aj_ref, w_ref, bias_col, C):
        acc = None
        for kd in range(3):
            blk = aj_ref[0:9 * C, kd * HW:kd * HW + S]
            c = jnp.dot(w_ref[kd], blk, preferred_element_type=jnp.float32)
            acc = c if acc is None else acc + c
        return acc + bias_col

    @pl.when(pl.program_id(0) == 0)
    def _init_pads():
        for aj_ref in a_refs:
            zero_pads(aj_ref)

    g1c, b1c = to_cols(e1_ref, g1_ref[...], b1_ref[...])
    g2c, b2c, bias1c, bias2c, brc = to_cols(
        e2_ref, g2_ref[...], b2_ref[...], bias1_ref[...], bias2_ref[...],
        br_ref[...])

    def emit_out(j, x, y):
        res = jnp.dot(wp_ref[...].astype(jnp.bfloat16),
                      x.astype(jnp.bfloat16),
                      preferred_element_type=jnp.float32)
        o_ref[j] = (y + res + brc).astype(o_ref.dtype)

    if NB == 2:
        x0, x1 = x_ref[0], x_ref[1]
        act1_0 = gn_silu(x0, p1_ref, g1c, b1c)
        stack_taps(a_refs[0], act1_0, Cin)
        act1_1 = gn_silu(x1, p1_ref, g1c, b1c)
        h0 = conv(a_refs[0], w1_ref, bias1c, Cin)
        stack_taps(a_refs[1], act1_1, Cin)
        act2_0 = gn_silu(h0, p2_ref, g2c, b2c)
        h1 = conv(a_refs[1], w1_ref, bias1c, Cin)
        stack_taps(a_refs[0], act2_0, Cout)
        act2_1 = gn_silu(h1, p2_ref, g2c, b2c)
        y0 = conv(a_refs[0], w2_ref, bias2c, Cout)
        stack_taps(a_refs[1], act2_1, Cout)
        emit_out(0, x0, y0)
        y1 = conv(a_refs[1], w2_ref, bias2c, Cout)
        emit_out(1, x1, y1)
    else:
        for j in range(NB):
            aj_ref = a_refs[j]
            x = x_ref[j]
            act1 = gn_silu(x, p1_ref, g1c, b1c)
            stack_taps(aj_ref, act1, Cin)
            h = conv(aj_ref, w1_ref, bias1c, Cin)
            act2 = gn_silu(h, p2_ref, g2c, b2c)
            stack_taps(aj_ref, act2, Cout)
            y = conv(aj_ref, w2_ref, bias2c, Cout)
            emit_out(j, x, y)


def kernel(x, g1, be1, w1, bias1, g2, be2, w2, bias2, wr, br):
    N, Cin, D, H, W = x.shape
    Cout = w1.shape[0]
    S = D * H * W
    HW = H * W

    xf = x.reshape(N, Cin, S)
    mhw = _hw_masks(D, H, W)
    p1 = _group_proj(Cin, _GROUPS, (Cin // _GROUPS) * S)
    p2 = _group_proj(Cout, _GROUPS, (Cout // _GROUPS) * S)
    w1f = _fold_w(w1)
    w2f = _fold_w(w2)
    g1v = g1.reshape(1, Cin)
    be1v = be1.reshape(1, Cin)
    g2v = g2.reshape(1, Cout)
    be2v = be2.reshape(1, Cout)
    bias1v = bias1.reshape(1, Cout)
    bias2v = bias2.reshape(1, Cout)
    wrf = wr.reshape(Cout, Cin)
    brv = br.reshape(1, Cout)
    e1 = jnp.asarray(np.eye(Cin, dtype=np.float32))
    e2 = jnp.asarray(np.eye(Cout, dtype=np.float32))
    NB = 2 if N % 2 == 0 else 1
    cnst = lambda *shape: pl.BlockSpec(shape, lambda n: (0,) * len(shape))
    out = pl.pallas_call(
        functools.partial(_block_kernel, shifts=_hw_shifts(H, W),
                          S=S, HW=HW, Cin=Cin, Cout=Cout, NB=NB),
        out_shape=jax.ShapeDtypeStruct((N, Cout, S), x.dtype),
        grid=(N // NB,),
        in_specs=[
            pl.BlockSpec((NB, Cin, S), lambda n: (n, 0, 0)),
            cnst(Cin, Cin),
            cnst(1, Cin),
            cnst(1, Cin),
            cnst(3, Cout, 9 * Cin),
            cnst(1, Cout),
            cnst(Cout, Cout),
            cnst(1, Cout),
            cnst(1, Cout),
            cnst(3, Cout, 9 * Cout),
            cnst(1, Cout),
            cnst(9, 1, S),
            cnst(Cout, Cin),
            cnst(1, Cout),
            cnst(Cin, Cin),
            cnst(Cout, Cout),
        ],
        out_specs=pl.BlockSpec((NB, Cout, S), lambda n: (n, 0, 0)),
        scratch_shapes=[pltpu.VMEM((9 * Cout, S + 2 * HW), jnp.bfloat16)
                        for _ in range(NB)],
        compiler_params=pltpu.CompilerParams(
            dimension_semantics=("parallel",)),
    )(xf, p1, g1v, be1v, w1f, bias1v, p2, g2v, be2v, w2f, bias2v,
      mhw, wrf, brv, e1, e2)

    return out.reshape(N, Cout, D, H, W)
